# baseline (speedup 1.0000x reference)
import functools

import jax
import jax.numpy as jnp
from jax import lax
from jax.experimental import pallas as pl
from jax.experimental.pallas import tpu as pltpu
from jax.experimental.pallas import tpu_sc as plsc

B = 16384
ED = 128
NF = 128

NC = 2
NS = 16
NW = NC * NS
CHUNK = 128
N_CHUNK = B // NW // CHUNK
N_IDX_ROWS = B // CHUNK


def _make_sc_gather_body(n_chunk, nbuf):
    nk = 2 * n_chunk

    def body(uid_hbm, iid_hbm, utab_hbm, itab_hbm,
             uout_hbm, iout_hbm, idx_v, rows_v, sem_g, sem_w):
        wid = lax.axis_index("s") * NC + lax.axis_index("c")
        r0 = wid * n_chunk

        pltpu.sync_copy(uid_hbm.at[pl.ds(r0, n_chunk)],
                        idx_v.at[pl.ds(0, n_chunk)])
        pltpu.sync_copy(iid_hbm.at[pl.ds(r0, n_chunk)],
                        idx_v.at[pl.ds(n_chunk, n_chunk)])

        srcs = [utab_hbm] * n_chunk + [itab_hbm] * n_chunk

        def dst(k):
            ref = uout_hbm if k < n_chunk else iout_hbm
            return ref.at[r0 + (k % n_chunk)]

        cps_g = [pltpu.async_copy(srcs[k].at[idx_v.at[k]], rows_v.at[k],
                                  sem_g)
                 for k in range(2)]
        cps_w = [None] * nk
        for k in range(nk):
            j = k + 2
            if j < nk:
                if j >= nbuf:
                    cps_w[j - nbuf].wait()
                cps_g.append(pltpu.async_copy(srcs[j].at[idx_v.at[j]],
                                              rows_v.at[j % nbuf], sem_g))
            cps_g[k].wait()
            cps_w[k] = pltpu.async_copy(rows_v.at[k % nbuf], dst(k), sem_w)
        for k in range(max(0, nk - nbuf), nk):
            cps_w[k].wait()

    return body


def _sc_gather(user_ids2d, item_ids2d, user_table, item_table):
    rows = user_ids2d.shape[0]
    n_chunk = rows // NW
    nbuf = min(6, 2 * n_chunk)
    mesh = plsc.VectorSubcoreMesh(core_axis_name="c", subcore_axis_name="s",
                                  num_cores=NC, num_subcores=NS)
    out_t = jax.ShapeDtypeStruct((rows, CHUNK, ED), jnp.float32)
    f = pl.kernel(
        _make_sc_gather_body(n_chunk, nbuf),
        out_type=(out_t, out_t),
        mesh=mesh,
        scratch_types=[
            pltpu.VMEM((2 * n_chunk, CHUNK), jnp.int32),
            pltpu.VMEM((nbuf, CHUNK, ED), jnp.float32),
            pltpu.SemaphoreType.DMA,
            pltpu.SemaphoreType.DMA,
        ],
    )
    return f(user_ids2d, item_ids2d, user_table, item_table)


def _content_body(x_ref, w1_ref, b1_ref, w2_ref, b2_ref, c_ref):
    bf = jnp.bfloat16
    f32 = jnp.float32
    x = x_ref[...].astype(bf)
    h = jnp.maximum(
        jnp.dot(x, w1_ref[...].astype(bf),
                preferred_element_type=f32) + b1_ref[...], 0.0)
    c = jnp.dot(h.astype(bf), w2_ref[...].astype(bf),
                preferred_element_type=f32) + b2_ref[...]
    c_ref[...] = c.astype(bf)


def _content(x, W1, b1, W2, b2, bs=4096):
    nblk = B // bs
    row_blk = lambda idx: (idx, 0)
    whole = lambda idx: (0, 0)
    return pl.pallas_call(
        _content_body,
        grid=(nblk,),
        in_specs=[
            pl.BlockSpec((bs, NF), row_blk),
            pl.BlockSpec((NF, ED), whole),
            pl.BlockSpec((1, ED), whole),
            pl.BlockSpec((ED, ED), whole),
            pl.BlockSpec((1, ED), whole),
        ],
        out_specs=pl.BlockSpec((bs, ED), row_blk),
        out_shape=jax.ShapeDtypeStruct((B, ED), jnp.bfloat16),
    )(x, W1, b1.reshape(1, ED), W2, b2.reshape(1, ED))


def _combine_body(u_ref, i_ref, c_ref, w3_ref, b3_ref, w4_ref, b4_ref,
                  o_ref):
    bf = jnp.bfloat16
    f32 = jnp.float32
    acc = (jnp.dot(u_ref[...].astype(bf), w3_ref[0:ED, :].astype(bf),
                   preferred_element_type=f32)
           + jnp.dot(i_ref[...].astype(bf), w3_ref[ED:2 * ED, :].astype(bf),
                     preferred_element_type=f32)
           + jnp.dot(c_ref[...], w3_ref[2 * ED:3 * ED, :].astype(bf),
                     preferred_element_type=f32)
           + b3_ref[...])
    p = jnp.maximum(acc, 0.0)
    z = jnp.dot(p.astype(bf), w4_ref[...].astype(bf),
                preferred_element_type=f32) + b4_ref[...]
    s = jax.nn.sigmoid(z)
    o_ref[...] = s.reshape(o_ref.shape)


def _combine_half(u, i, c, W3, b3, W4, b4, half, bs=4096):
    hb = B // 2
    nblk = hb // bs
    off = half * hb // bs
    row_blk = lambda idx: (idx, 0)
    c_blk = lambda idx: (idx + off, 0)
    whole = lambda idx: (0, 0)
    return pl.pallas_call(
        _combine_body,
        grid=(nblk,),
        in_specs=[
            pl.BlockSpec((bs, ED), row_blk),
            pl.BlockSpec((bs, ED), row_blk),
            pl.BlockSpec((bs, ED), c_blk),
            pl.BlockSpec((3 * ED, ED), whole),
            pl.BlockSpec((1, ED), whole),
            pl.BlockSpec((ED, 1), whole),
            pl.BlockSpec((1, 1), whole),
        ],
        out_specs=pl.BlockSpec((bs // 128, 128), row_blk),
        out_shape=jax.ShapeDtypeStruct((hb // 128, 128), jnp.float32),
    )(u, i, c, W3, b3.reshape(1, ED), W4, b4.reshape(1, 1))


def kernel(user_ids, item_ids, item_features, user_table, item_table,
           W1, b1, W2, b2, W3, b3, W4, b4):
    uid2 = user_ids.astype(jnp.int32).reshape(N_IDX_ROWS, CHUNK)
    iid2 = item_ids.astype(jnp.int32).reshape(N_IDX_ROWS, CHUNK)
    hr = N_IDX_ROWS // 2
    hb = B // 2
    u3a, i3a = _sc_gather(uid2[:hr], iid2[:hr], user_table, item_table)
    u3b, i3b = _sc_gather(uid2[hr:], iid2[hr:], user_table, item_table)
    c = _content(item_features, W1, b1, W2, b2)
    oa = _combine_half(u3a.reshape(hb, ED), i3a.reshape(hb, ED), c,
                       W3, b3, W4, b4, half=0)
    ob = _combine_half(u3b.reshape(hb, ED), i3b.reshape(hb, ED), c,
                       W3, b3, W4, b4, half=1)
    return jnp.concatenate([oa.reshape(hb, 1), ob.reshape(hb, 1)], axis=0)
